# baseline (speedup 1.0000x reference)
import functools

import jax
import jax.numpy as jnp
from jax import lax
from jax.experimental import pallas as pl
from jax.experimental.pallas import tpu as pltpu
from jax.experimental.pallas import tpu_sc as plsc

_D = 128
_E = 320000
_N = 10000
_C = 80
_NW = 32
_EPW = _E // _NW
_CPW = _EPW // _C
_R = 4
_LAG = 3
_RL = 4
_ILAG = 1
_T = (_CPW - 1) // _R
_KH = 2
_TROWS = 640

_mesh = plsc.VectorSubcoreMesh(core_axis_name="c", subcore_axis_name="s")


@functools.partial(
    pl.kernel,
    out_type=jax.ShapeDtypeStruct((_E, _D), jnp.float32),
    mesh=_mesh,
    scratch_types=[
        pltpu.VMEM((_R, _C), jnp.int32),
        pltpu.VMEM((_R, _C, _D), jnp.float32),
        pltpu.VMEM_SHARED((_N, _D), jnp.float32),
        pltpu.SemaphoreType.DMA((_R,)),
        pltpu.SemaphoreType.DMA((_R,)),
        pltpu.SemaphoreType.DMA((_R,)),
        pltpu.SemaphoreType.DMA,
    ],
)
def _gather_kernel(table, idx_hbm, out, idxr, rbuf, shtab, isem, gsem, wsem,
                   ssem):
    sid = lax.axis_index("s")
    wid = sid * 2 + lax.axis_index("c")
    e0 = wid * _EPW

    def start_idx(c, slot):
        pltpu.async_copy(
            idx_hbm.at[pl.ds(e0 + c * _C, _C)], idxr.at[slot], isem.at[slot]
        )

    def wait_idx(slot):
        pltpu.make_async_copy(
            idx_hbm.at[pl.ds(0, _C)], idxr.at[slot], isem.at[slot]
        ).wait()

    def start_gather(slot, src):
        pltpu.async_copy(src.at[idxr.at[slot]], rbuf.at[slot], gsem.at[slot])

    def wait_gather(slot):
        pltpu.make_async_copy(
            shtab.at[idxr.at[0]], rbuf.at[slot], gsem.at[slot]
        ).wait()

    def start_write(c, slot):
        pltpu.async_copy(
            rbuf.at[slot], out.at[pl.ds(e0 + c * _C, _C)], wsem.at[slot]
        )

    def wait_write(slot):
        pltpu.make_async_copy(
            rbuf.at[slot], out.at[pl.ds(0, _C)], wsem.at[slot]
        ).wait()

    for s in range(_RL):
        start_idx(s, s)

    off = jnp.where(sid == 15, _N - _TROWS, sid * _TROWS)
    off = pl.multiple_of(off, 16)
    pltpu.async_copy(
        table.at[pl.ds(off, _TROWS)], shtab.at[pl.ds(off, _TROWS)], ssem
    )

    for s in range(_ILAG):
        wait_idx(s)
        start_gather(s, table)

    def make_outer(src):
        def outer(t, carry):
            for b in range(_R):
                c = t * _R + b
                slot = b
                slot_w = (b + _R - _LAG) % _R
                slot_g = (b + _ILAG) % _R

                if b >= _LAG:
                    wait_write(slot_w)
                else:
                    @pl.when(t > 0)
                    def _():
                        wait_write(slot_w)

                wait_idx(slot_g)
                start_gather(slot_g, src)

                wait_gather(slot)
                start_write(c, slot)

                if b == 0:
                    start_idx(c + _RL, slot)
                else:
                    @pl.when(t < _T - 1)
                    def _():
                        start_idx(c + _RL, slot)
            return carry
        return outer

    lax.fori_loop(0, _KH, make_outer(table), 0)
    pltpu.make_async_copy(
        table.at[pl.ds(off, _TROWS)], shtab.at[pl.ds(off, _TROWS)], ssem
    ).wait()
    plsc.subcore_barrier()
    lax.fori_loop(_KH, _T, make_outer(shtab), 0)

    wait_gather(0)
    start_write(_CPW - 1, 0)
    wait_write(1)
    wait_write(2)
    wait_write(3)
    wait_write(0)


def kernel(src_node_states, dst_node_states, dst_index, src_index, edge_states):
    del dst_node_states, dst_index, edge_states
    return _gather_kernel(src_node_states, src_index)


# speedup vs baseline: 1.0059x; 1.0059x over previous
import functools

import jax
import jax.numpy as jnp
from jax import lax
from jax.experimental import pallas as pl
from jax.experimental.pallas import tpu as pltpu
from jax.experimental.pallas import tpu_sc as plsc

_D = 128
_E = 320000
_N = 10000
_C = 80
_NW = 32
_EPW = _E // _NW
_CPW = _EPW // _C
_R = 4
_LAG = 2
_RL = 4
_ILAG = 2
_T = (_CPW - 1) // _R
_KH = 2
_TROWS = 640

_mesh = plsc.VectorSubcoreMesh(core_axis_name="c", subcore_axis_name="s")


@functools.partial(
    pl.kernel,
    out_type=jax.ShapeDtypeStruct((_E, _D), jnp.float32),
    mesh=_mesh,
    scratch_types=[
        pltpu.VMEM((_R, _C), jnp.int32),
        pltpu.VMEM((_R, _C, _D), jnp.float32),
        pltpu.VMEM_SHARED((_N, _D), jnp.float32),
        pltpu.SemaphoreType.DMA((_R,)),
        pltpu.SemaphoreType.DMA((_R,)),
        pltpu.SemaphoreType.DMA((_R,)),
        pltpu.SemaphoreType.DMA,
    ],
)
def _gather_kernel(table, idx_hbm, out, idxr, rbuf, shtab, isem, gsem, wsem,
                   ssem):
    sid = lax.axis_index("s")
    wid = sid * 2 + lax.axis_index("c")
    e0 = wid * _EPW

    def start_idx(c, slot):
        pltpu.async_copy(
            idx_hbm.at[pl.ds(e0 + c * _C, _C)], idxr.at[slot], isem.at[slot]
        )

    def wait_idx(slot):
        pltpu.make_async_copy(
            idx_hbm.at[pl.ds(0, _C)], idxr.at[slot], isem.at[slot]
        ).wait()

    def start_gather(slot, src):
        pltpu.async_copy(src.at[idxr.at[slot]], rbuf.at[slot], gsem.at[slot])

    def wait_gather(slot):
        pltpu.make_async_copy(
            shtab.at[idxr.at[0]], rbuf.at[slot], gsem.at[slot]
        ).wait()

    def start_write(c, slot):
        pltpu.async_copy(
            rbuf.at[slot], out.at[pl.ds(e0 + c * _C, _C)], wsem.at[slot]
        )

    def wait_write(slot):
        pltpu.make_async_copy(
            rbuf.at[slot], out.at[pl.ds(0, _C)], wsem.at[slot]
        ).wait()

    for s in range(_RL):
        start_idx(s, s)

    off = jnp.where(sid == 15, _N - _TROWS, sid * _TROWS)
    off = pl.multiple_of(off, 16)
    pltpu.async_copy(
        table.at[pl.ds(off, _TROWS)], shtab.at[pl.ds(off, _TROWS)], ssem
    )

    for s in range(_ILAG):
        wait_idx(s)
        start_gather(s, table)

    def make_outer(src):
        def outer(t, carry):
            for b in range(_R):
                c = t * _R + b
                slot = b
                slot_w = (b + _R - _LAG) % _R
                slot_g = (b + _ILAG) % _R

                if b >= _LAG:
                    wait_write(slot_w)
                else:
                    @pl.when(t > 0)
                    def _():
                        wait_write(slot_w)

                if b == _R - 1:
                    @pl.when(t < _T - 1)
                    def _():
                        wait_idx(slot_g)
                        start_gather(slot_g, src)
                else:
                    wait_idx(slot_g)
                    start_gather(slot_g, src)

                wait_gather(slot)
                start_write(c, slot)

                if b == 0:
                    start_idx(c + _RL, slot)
                else:
                    @pl.when(t < _T - 1)
                    def _():
                        start_idx(c + _RL, slot)
            return carry
        return outer

    lax.fori_loop(0, _KH, make_outer(table), 0)
    pltpu.make_async_copy(
        table.at[pl.ds(off, _TROWS)], shtab.at[pl.ds(off, _TROWS)], ssem
    ).wait()
    plsc.subcore_barrier()
    lax.fori_loop(_KH, _T, make_outer(shtab), 0)

    wait_gather(0)
    start_write(_CPW - 1, 0)
    wait_write(2)
    wait_write(3)
    wait_write(0)


def kernel(src_node_states, dst_node_states, dst_index, src_index, edge_states):
    del dst_node_states, dst_index, edge_states
    return _gather_kernel(src_node_states, src_index)
